# Initial kernel scaffold; baseline (speedup 1.0000x reference)
import jax
import jax.numpy as jnp
from jax.experimental import pallas as pl


def kernel(x, edge_index, batch, W1, b1, W2, b2, Wl, bl):
    raise NotImplementedError("write your pallas kernel here")


# speedup vs baseline: 11.7309x; 11.7309x over previous
import functools

import jax
import jax.numpy as jnp
from jax import lax
from jax.experimental import pallas as pl
from jax.experimental.pallas import tpu as pltpu
from jax.experimental.pallas import tpu_sc as plsc

N = 10000
E = 160000
F_IN = 256
H = 512
C = 10
G = 64

NC = 2
NS = 16
L = 16

EB = 80
EROWS = E // EB
RPT = EROWS // NS
FC = 128

ROW8 = 624
TAIL = N - NS * ROW8
TOFF = NS * ROW8
ZR = 16

NB = 2000
GRID = N // NB

_f32 = jnp.float32


def _mesh():
    return plsc.VectorSubcoreMesh(
        core_axis_name="c", subcore_axis_name="s", num_cores=NC, num_subcores=NS
    )


def _zero_zbuf(zb, rows, width):
    zero = jnp.zeros((L,), _f32)

    def zrow(r, _):
        for k in range(width // L):
            zb[r, pl.ds(k * L, L)] = zero
        return 0

    lax.fori_loop(0, rows, zrow, 0)


def _clear_shared(zb, shared, s):
    for t in range(ROW8 // ZR):
        pltpu.sync_copy(zb, shared.at[pl.ds(s * ROW8 + t * ZR, ZR)])

    @pl.when(s == 0)
    def _():
        pltpu.sync_copy(zb.at[pl.ds(0, TAIL)], shared.at[pl.ds(TOFF, TAIL)])


def _dump_shared(shared, out, s):
    pltpu.sync_copy(shared.at[pl.ds(s * ROW8, ROW8)],
                    out.at[pl.ds(s * ROW8, ROW8)])

    @pl.when(s == 0)
    def _():
        pltpu.sync_copy(shared.at[pl.ds(TOFF, TAIL)],
                        out.at[pl.ds(TOFF, TAIL)])


def _deg_body(dst_hbm, out_hbm, shared, dst_v, zb, ones_v):
    c = lax.axis_index("c")
    s = lax.axis_index("s")

    @pl.when(c == 0)
    def _():
        _zero_zbuf(zb, ZR, FC)
        one = jnp.full((L,), 1.0, _f32)

        def orow(r, _):
            for k in range(FC // L):
                ones_v[r, pl.ds(k * L, L)] = one
            return 0

        lax.fori_loop(0, EB, orow, 0)
        _clear_shared(zb, shared, s)
        plsc.subcore_barrier()
        pltpu.sync_copy(dst_hbm.at[s], dst_v)

        def body(j, _):
            pltpu.sync_copy(ones_v, shared.at[dst_v.at[j]], add=True)
            return 0

        lax.fori_loop(0, RPT, body, 0)
        plsc.subcore_barrier()
        _dump_shared(shared, out_hbm, s)


_deg = functools.partial(
    pl.kernel,
    out_type=jax.ShapeDtypeStruct((N, FC), _f32),
    mesh=_mesh(),
    scratch_types=[
        pltpu.VMEM_SHARED((N, FC), _f32),
        pltpu.VMEM((RPT, EB), jnp.int32),
        pltpu.VMEM((ZR, FC), _f32),
        pltpu.VMEM((EB, FC), _f32),
    ],
)(_deg_body)


def _make_agg(n_chunks):
    out_t = tuple(jax.ShapeDtypeStruct((N, FC), _f32) for _ in range(n_chunks))
    scratch = [
        pltpu.VMEM_SHARED((N, FC), _f32),
        pltpu.VMEM((RPT, EB), jnp.int32),
        pltpu.VMEM((RPT, EB), jnp.int32),
        pltpu.VMEM((EB, FC), _f32),
        pltpu.VMEM((ZR, FC), _f32),
        pltpu.SemaphoreType.DMA,
    ]

    def body(src_hbm, dst_hbm, *rest):
        xs = rest[:n_chunks]
        outs = rest[n_chunks:2 * n_chunks]
        shared, src_v, dst_v, rows_v, zb, sem = rest[2 * n_chunks:]
        c = lax.axis_index("c")
        s = lax.axis_index("s")

        _zero_zbuf(zb, ZR, FC)
        pltpu.sync_copy(src_hbm.at[s], src_v)
        pltpu.sync_copy(dst_hbm.at[s], dst_v)

        for k in range(n_chunks):
            @pl.when(c == (k % NC))
            def _(k=k):
                _clear_shared(zb, shared, s)
                plsc.subcore_barrier()

                def ebody(j, _):
                    pltpu.async_copy(xs[k].at[src_v.at[j]], rows_v, sem).wait()
                    pltpu.sync_copy(rows_v, shared.at[dst_v.at[j]], add=True)
                    return 0

                lax.fori_loop(0, RPT, ebody, 0)
                plsc.subcore_barrier()
                _dump_shared(shared, outs[k], s)

    return functools.partial(
        pl.kernel, out_type=out_t, mesh=_mesh(), scratch_types=scratch
    )(body)


_agg256 = _make_agg(2)
_agg512 = _make_agg(4)


def _prescale_body(deg_ref, x_ref, xs0_ref, xs1_ref, dinv_ref):
    d = deg_ref[:, 0:1] + 1.0
    dinv = lax.rsqrt(d)
    dinv_ref[...] = dinv
    xs = x_ref[...] * dinv
    xs0_ref[...] = xs[:, :FC]
    xs1_ref[...] = xs[:, FC:]


def _prescale(deg2, x):
    return pl.pallas_call(
        _prescale_body,
        grid=(GRID,),
        in_specs=[
            pl.BlockSpec((NB, FC), lambda i: (i, 0)),
            pl.BlockSpec((NB, F_IN), lambda i: (i, 0)),
        ],
        out_specs=[
            pl.BlockSpec((NB, FC), lambda i: (i, 0)),
            pl.BlockSpec((NB, FC), lambda i: (i, 0)),
            pl.BlockSpec((NB, 1), lambda i: (i, 0)),
        ],
        out_shape=[
            jax.ShapeDtypeStruct((N, FC), _f32),
            jax.ShapeDtypeStruct((N, FC), _f32),
            jax.ShapeDtypeStruct((N, 1), _f32),
        ],
    )(deg2, x)


def _layer1_body(s0, s1, x0, x1, dinv, w, b, h0, h1, h2, h3):
    dv = dinv[...]
    a0 = dv * (s0[...] + x0[...])
    a1 = dv * (s1[...] + x1[...])
    agg = jnp.concatenate([a0, a1], axis=1)
    h = lax.dot_general(agg, w[...], (((1,), (0,)), ((), ())),
                        preferred_element_type=_f32)
    h = jnp.maximum(h + b[...], 0.0) * dv
    h0[...] = h[:, 0 * FC:1 * FC]
    h1[...] = h[:, 1 * FC:2 * FC]
    h2[...] = h[:, 2 * FC:3 * FC]
    h3[...] = h[:, 3 * FC:4 * FC]


def _layer1(s0, s1, x0, x1, dinv, w1, b1):
    return pl.pallas_call(
        _layer1_body,
        grid=(GRID,),
        in_specs=[
            pl.BlockSpec((NB, FC), lambda i: (i, 0)),
            pl.BlockSpec((NB, FC), lambda i: (i, 0)),
            pl.BlockSpec((NB, FC), lambda i: (i, 0)),
            pl.BlockSpec((NB, FC), lambda i: (i, 0)),
            pl.BlockSpec((NB, 1), lambda i: (i, 0)),
            pl.BlockSpec((F_IN, H), lambda i: (0, 0)),
            pl.BlockSpec((1, H), lambda i: (0, 0)),
        ],
        out_specs=[pl.BlockSpec((NB, FC), lambda i: (i, 0))] * 4,
        out_shape=[jax.ShapeDtypeStruct((N, FC), _f32)] * 4,
    )(s0, s1, x0, x1, dinv, w1, b1)


def _layer2_body(s0, s1, s2, s3, h0, h1, h2, h3, dinv, w, b, bat, wl, bl,
                 out_ref, sums, counts):
    i = pl.program_id(0)

    @pl.when(i == 0)
    def _():
        sums[...] = jnp.zeros_like(sums)
        counts[...] = jnp.zeros_like(counts)

    dv = dinv[...]
    agg = jnp.concatenate(
        [dv * (s0[...] + h0[...]), dv * (s1[...] + h1[...]),
         dv * (s2[...] + h2[...]), dv * (s3[...] + h3[...])], axis=1)
    h = lax.dot_general(agg, w[...], (((1,), (0,)), ((), ())),
                        preferred_element_type=_f32)
    h = jnp.maximum(h + b[...], 0.0)

    gid = lax.broadcasted_iota(jnp.int32, (NB, G), 1)
    oh = (bat[...] == gid).astype(_f32)
    sums[...] = sums[...] + lax.dot_general(
        oh, h, (((0,), (0,)), ((), ())), preferred_element_type=_f32)
    ones = jnp.ones((NB, 1), _f32)
    counts[:, 0:1] = counts[:, 0:1] + lax.dot_general(
        oh, ones, (((0,), (0,)), ((), ())), preferred_element_type=_f32)

    @pl.when(i == GRID - 1)
    def _():
        pooled = sums[...] / jnp.maximum(counts[:, 0:1], 1.0)
        out_ref[...] = lax.dot_general(
            pooled, wl[...], (((1,), (0,)), ((), ())),
            preferred_element_type=_f32) + bl[...]


def _layer2(s, hs, dinv, w2, b2, bat2, wl, bl):
    return pl.pallas_call(
        _layer2_body,
        grid=(GRID,),
        in_specs=[pl.BlockSpec((NB, FC), lambda i: (i, 0))] * 8 + [
            pl.BlockSpec((NB, 1), lambda i: (i, 0)),
            pl.BlockSpec((H, H), lambda i: (0, 0)),
            pl.BlockSpec((1, H), lambda i: (0, 0)),
            pl.BlockSpec((NB, 1), lambda i: (i, 0)),
            pl.BlockSpec((H, C), lambda i: (0, 0)),
            pl.BlockSpec((1, C), lambda i: (0, 0)),
        ],
        out_specs=pl.BlockSpec((G, C), lambda i: (0, 0)),
        out_shape=jax.ShapeDtypeStruct((G, C), _f32),
        scratch_shapes=[pltpu.VMEM((G, H), _f32), pltpu.VMEM((G, 128), _f32)],
    )(*s, *hs, dinv, w2, b2, bat2, wl, bl)


def kernel(x, edge_index, batch, W1, b1, W2, b2, Wl, bl):
    src3 = edge_index[0].reshape(NS, RPT, EB)
    dst3 = edge_index[1].reshape(NS, RPT, EB)

    deg2 = _deg(dst3)
    xs0, xs1, dinv = _prescale(deg2, x)
    S0, S1 = _agg256(src3, dst3, xs0, xs1)
    hs = _layer1(S0, S1, xs0, xs1, dinv, W1, b1.reshape(1, H))
    T = _agg512(src3, dst3, *hs)
    return _layer2(T, hs, dinv, W2, b2.reshape(1, H), batch.reshape(N, 1),
                   Wl, bl.reshape(1, C))


# speedup vs baseline: 18.1739x; 1.5492x over previous
import functools

import jax
import jax.numpy as jnp
from jax import lax
from jax.experimental import pallas as pl
from jax.experimental.pallas import tpu as pltpu
from jax.experimental.pallas import tpu_sc as plsc

N = 10000
E = 160000
F_IN = 256
H = 512
C = 10
G = 64

NC = 2
NS = 16
L = 16

EB = 80
EROWS = E // EB
RPT = EROWS // NS
FC = 128

ROW8 = 624
TAIL = N - NS * ROW8
TOFF = NS * ROW8
ZR = 16

NB = 2000
GRID = N // NB

_f32 = jnp.float32


def _mesh():
    return plsc.VectorSubcoreMesh(
        core_axis_name="c", subcore_axis_name="s", num_cores=NC, num_subcores=NS
    )


def _zero_zbuf(zb, rows, width):
    zero = jnp.zeros((L,), _f32)

    def zrow(r, _):
        for k in range(width // L):
            zb[r, pl.ds(k * L, L)] = zero
        return 0

    lax.fori_loop(0, rows, zrow, 0)


def _clear_shared(zb, shared, s):
    for t in range(ROW8 // ZR):
        pltpu.sync_copy(zb, shared.at[pl.ds(s * ROW8 + t * ZR, ZR)])

    @pl.when(s == 0)
    def _():
        pltpu.sync_copy(zb.at[pl.ds(0, TAIL)], shared.at[pl.ds(TOFF, TAIL)])


def _dump_shared(shared, out, s):
    pltpu.sync_copy(shared.at[pl.ds(s * ROW8, ROW8)],
                    out.at[pl.ds(s * ROW8, ROW8)])

    @pl.when(s == 0)
    def _():
        pltpu.sync_copy(shared.at[pl.ds(TOFF, TAIL)],
                        out.at[pl.ds(TOFF, TAIL)])


def _deg_body(dst_hbm, out_hbm, shared, dst_v, zb, ones_v):
    c = lax.axis_index("c")
    s = lax.axis_index("s")

    @pl.when(c == 0)
    def _():
        _zero_zbuf(zb, ZR, FC)
        one = jnp.full((L,), 1.0, _f32)

        def orow(r, _):
            for k in range(FC // L):
                ones_v[r, pl.ds(k * L, L)] = one
            return 0

        lax.fori_loop(0, EB, orow, 0)
        _clear_shared(zb, shared, s)
        plsc.subcore_barrier()
        pltpu.sync_copy(dst_hbm.at[s], dst_v)

        def body(j, _):
            pltpu.sync_copy(ones_v, shared.at[dst_v.at[j]], add=True)
            return 0

        lax.fori_loop(0, RPT, body, 0)
        plsc.subcore_barrier()
        _dump_shared(shared, out_hbm, s)


_deg = functools.partial(
    pl.kernel,
    out_type=jax.ShapeDtypeStruct((N, FC), _f32),
    mesh=_mesh(),
    scratch_types=[
        pltpu.VMEM_SHARED((N, FC), _f32),
        pltpu.VMEM((RPT, EB), jnp.int32),
        pltpu.VMEM((ZR, FC), _f32),
        pltpu.VMEM((EB, FC), _f32),
    ],
)(_deg_body)


EPT = E // NS


def _make_agg(n_chunks):
    out_t = tuple(jax.ShapeDtypeStruct((N, FC), _f32) for _ in range(n_chunks))
    scratch = [
        pltpu.VMEM_SHARED((N, FC), _f32),
        pltpu.VMEM((EPT,), jnp.int32),
        pltpu.VMEM((RPT, EB), jnp.int32),
        pltpu.VMEM((EB, FC), _f32),
        pltpu.VMEM((EB, FC), _f32),
        pltpu.SemaphoreType.DMA,
        pltpu.SemaphoreType.DMA,
    ]

    def body(src_hbm, dst_hbm, *rest):
        xs = rest[:n_chunks]
        outs = rest[n_chunks:2 * n_chunks]
        shared, src_v, dst_v, r0, r1, sem0, sem1 = rest[2 * n_chunks:]
        c = lax.axis_index("c")
        s = lax.axis_index("s")

        _zero_zbuf(r0, EB, FC)
        pltpu.sync_copy(src_hbm.at[pl.ds(s * EPT, EPT)], src_v)
        pltpu.sync_copy(dst_hbm.at[s], dst_v)

        for k in range(n_chunks):
            @pl.when(c == (k % NC))
            def _(k=k):
                for t in range(7):
                    pltpu.sync_copy(r0, shared.at[pl.ds(s * ROW8 + t * EB, EB)])
                pltpu.sync_copy(r0.at[pl.ds(0, ROW8 - 7 * EB)],
                                shared.at[pl.ds(s * ROW8 + 7 * EB,
                                                ROW8 - 7 * EB)])

                @pl.when(s == 0)
                def _():
                    pltpu.sync_copy(r0.at[pl.ds(0, TAIL)],
                                    shared.at[pl.ds(TOFF, TAIL)])

                plsc.subcore_barrier()

                def gather(j, buf, sem):
                    return pltpu.async_copy(
                        xs[k].at[src_v.at[pl.ds(j * EB, EB)]], buf, sem)

                gather(0, r0, sem0)

                def ebody(it, _):
                    j0 = it * 2
                    gather(j0 + 1, r1, sem1)
                    pltpu.make_async_copy(
                        xs[k].at[src_v.at[pl.ds(0, EB)]], r0, sem0).wait()
                    pltpu.sync_copy(r0, shared.at[dst_v.at[j0]], add=True)
                    gather(j0 + 2, r0, sem0)
                    pltpu.make_async_copy(
                        xs[k].at[src_v.at[pl.ds(0, EB)]], r1, sem1).wait()
                    pltpu.sync_copy(r1, shared.at[dst_v.at[j0 + 1]], add=True)
                    return 0

                lax.fori_loop(0, (RPT - 1) // 2, ebody, 0)
                pltpu.make_async_copy(
                    xs[k].at[src_v.at[pl.ds(0, EB)]], r0, sem0).wait()
                pltpu.sync_copy(r0, shared.at[dst_v.at[RPT - 1]], add=True)
                plsc.subcore_barrier()
                _dump_shared(shared, outs[k], s)

    return functools.partial(
        pl.kernel, out_type=out_t, mesh=_mesh(), scratch_types=scratch
    )(body)


_agg256 = _make_agg(2)
_agg512 = _make_agg(4)


def _prescale_body(deg_ref, x_ref, xs0_ref, xs1_ref, dinv_ref):
    d = deg_ref[:, 0:1] + 1.0
    dinv = lax.rsqrt(d)
    dinv_ref[...] = dinv
    xs = x_ref[...] * dinv
    xs0_ref[...] = xs[:, :FC]
    xs1_ref[...] = xs[:, FC:]


def _prescale(deg2, x):
    return pl.pallas_call(
        _prescale_body,
        grid=(GRID,),
        in_specs=[
            pl.BlockSpec((NB, FC), lambda i: (i, 0)),
            pl.BlockSpec((NB, F_IN), lambda i: (i, 0)),
        ],
        out_specs=[
            pl.BlockSpec((NB, FC), lambda i: (i, 0)),
            pl.BlockSpec((NB, FC), lambda i: (i, 0)),
            pl.BlockSpec((NB, 1), lambda i: (i, 0)),
        ],
        out_shape=[
            jax.ShapeDtypeStruct((N, FC), _f32),
            jax.ShapeDtypeStruct((N, FC), _f32),
            jax.ShapeDtypeStruct((N, 1), _f32),
        ],
    )(deg2, x)


def _layer1_body(s0, s1, x0, x1, dinv, w, b, h0, h1, h2, h3):
    dv = dinv[...]
    a0 = dv * (s0[...] + x0[...])
    a1 = dv * (s1[...] + x1[...])
    agg = jnp.concatenate([a0, a1], axis=1)
    h = lax.dot_general(agg, w[...], (((1,), (0,)), ((), ())),
                        preferred_element_type=_f32)
    h = jnp.maximum(h + b[...], 0.0) * dv
    h0[...] = h[:, 0 * FC:1 * FC]
    h1[...] = h[:, 1 * FC:2 * FC]
    h2[...] = h[:, 2 * FC:3 * FC]
    h3[...] = h[:, 3 * FC:4 * FC]


def _layer1(s0, s1, x0, x1, dinv, w1, b1):
    return pl.pallas_call(
        _layer1_body,
        grid=(GRID,),
        in_specs=[
            pl.BlockSpec((NB, FC), lambda i: (i, 0)),
            pl.BlockSpec((NB, FC), lambda i: (i, 0)),
            pl.BlockSpec((NB, FC), lambda i: (i, 0)),
            pl.BlockSpec((NB, FC), lambda i: (i, 0)),
            pl.BlockSpec((NB, 1), lambda i: (i, 0)),
            pl.BlockSpec((F_IN, H), lambda i: (0, 0)),
            pl.BlockSpec((1, H), lambda i: (0, 0)),
        ],
        out_specs=[pl.BlockSpec((NB, FC), lambda i: (i, 0))] * 4,
        out_shape=[jax.ShapeDtypeStruct((N, FC), _f32)] * 4,
    )(s0, s1, x0, x1, dinv, w1, b1)


def _layer2_body(s0, s1, s2, s3, h0, h1, h2, h3, dinv, w, b, bat, wl, bl,
                 out_ref, sums, counts):
    i = pl.program_id(0)

    @pl.when(i == 0)
    def _():
        sums[...] = jnp.zeros_like(sums)
        counts[...] = jnp.zeros_like(counts)

    dv = dinv[...]
    agg = jnp.concatenate(
        [dv * (s0[...] + h0[...]), dv * (s1[...] + h1[...]),
         dv * (s2[...] + h2[...]), dv * (s3[...] + h3[...])], axis=1)
    h = lax.dot_general(agg, w[...], (((1,), (0,)), ((), ())),
                        preferred_element_type=_f32)
    h = jnp.maximum(h + b[...], 0.0)

    gid = lax.broadcasted_iota(jnp.int32, (NB, G), 1)
    oh = (bat[...] == gid).astype(_f32)
    sums[...] = sums[...] + lax.dot_general(
        oh, h, (((0,), (0,)), ((), ())), preferred_element_type=_f32)
    ones = jnp.ones((NB, 1), _f32)
    counts[:, 0:1] = counts[:, 0:1] + lax.dot_general(
        oh, ones, (((0,), (0,)), ((), ())), preferred_element_type=_f32)

    @pl.when(i == GRID - 1)
    def _():
        pooled = sums[...] / jnp.maximum(counts[:, 0:1], 1.0)
        out_ref[...] = lax.dot_general(
            pooled, wl[...], (((1,), (0,)), ((), ())),
            preferred_element_type=_f32) + bl[...]


def _layer2(s, hs, dinv, w2, b2, bat2, wl, bl):
    return pl.pallas_call(
        _layer2_body,
        grid=(GRID,),
        in_specs=[pl.BlockSpec((NB, FC), lambda i: (i, 0))] * 8 + [
            pl.BlockSpec((NB, 1), lambda i: (i, 0)),
            pl.BlockSpec((H, H), lambda i: (0, 0)),
            pl.BlockSpec((1, H), lambda i: (0, 0)),
            pl.BlockSpec((NB, 1), lambda i: (i, 0)),
            pl.BlockSpec((H, C), lambda i: (0, 0)),
            pl.BlockSpec((1, C), lambda i: (0, 0)),
        ],
        out_specs=pl.BlockSpec((G, C), lambda i: (0, 0)),
        out_shape=jax.ShapeDtypeStruct((G, C), _f32),
        scratch_shapes=[pltpu.VMEM((G, H), _f32), pltpu.VMEM((G, 128), _f32)],
    )(*s, *hs, dinv, w2, b2, bat2, wl, bl)


def kernel(x, edge_index, batch, W1, b1, W2, b2, Wl, bl):
    src1 = edge_index[0]
    dst3 = edge_index[1].reshape(NS, RPT, EB)

    deg2 = _deg(dst3)
    xs0, xs1, dinv = _prescale(deg2, x)
    S0, S1 = _agg256(src1, dst3, xs0, xs1)
    hs = _layer1(S0, S1, xs0, xs1, dinv, W1, b1.reshape(1, H))
    T = _agg512(src1, dst3, *hs)
    return _layer2(T, hs, dinv, W2, b2.reshape(1, H), batch.reshape(N, 1),
                   Wl, bl.reshape(1, C))
